# baseline (speedup 1.0000x reference)
import functools

import jax
import jax.numpy as jnp
from jax import lax
from jax.experimental import pallas as pl
from jax.experimental.pallas import tpu as pltpu
from jax.experimental.pallas import tpu_sc as plsc

D = 64
DP = 128
SP = 136
SCALE = 8.0
L = 16
NBUF = 8
NST = 4


def _build_sc_kernel(B: int, S: int, V: int):
    info = plsc.get_sparse_core_info()
    NW = info.num_cores * info.num_subcores
    CH = B // NW
    assert CH == 128 and S % NBUF == 0 and D % L == 0

    mesh = plsc.VectorSubcoreMesh(core_axis_name="c", subcore_axis_name="s")

    @functools.partial(
        pl.kernel,
        mesh=mesh,
        out_type=jax.ShapeDtypeStruct((S, D // 8, NW, 8, DP), jnp.float32),
        scratch_types=[
            pltpu.VMEM((S, CH), jnp.int32),
            *[pltpu.VMEM((CH, D), jnp.float32) for _ in range(NBUF)],
            *[pltpu.VMEM((D, SP), jnp.float32) for _ in range(NST)],
            *[pltpu.SemaphoreType.DMA for _ in range(NBUF + NST)],
        ],
        compiler_params=pltpu.CompilerParams(
            use_tc_tiling_on_sc=False, needs_layout_passes=False),
    )
    def k(xt_hbm, tp_hbm, out_hbm, idx_v, *refs):
        emb = refs[:NBUF]
        st = refs[NBUF:NBUF + NST]
        gsem = refs[NBUF + NST:2 * NBUF + NST]
        osem = refs[2 * NBUF + NST:]

        wid = lax.axis_index("s") * info.num_cores + lax.axis_index("c")

        pltpu.sync_copy(xt_hbm.at[pl.ds(0, S), pl.ds(wid * CH, CH)], idx_v)

        dvecs = [lax.iota(jnp.int32, L) + L * kk for kk in range(D // L)]

        def fire_gather(s, b):
            pltpu.async_copy(tp_hbm.at[idx_v.at[s]], emb[b], gsem[b])

        def drain_gather(b):
            pltpu.make_async_copy(tp_hbm.at[pl.ds(0, CH)], emb[b], gsem[b]).wait()

        def fire_out(s, p):
            for do in range(D // 8):
                pltpu.async_copy(
                    st[p].at[pl.ds(8 * do, 8), pl.ds(0, DP)],
                    out_hbm.at[s, do, wid], osem[p])

        def wait_out(p):
            for do in range(D // 8):
                pltpu.make_async_copy(
                    st[p].at[pl.ds(0, 8), pl.ds(0, DP)],
                    out_hbm.at[0, 0, 0], osem[p]).wait()

        def transpose_scale(b, p):
            @plsc.parallel_loop(0, CH, unroll=2)
            def body(bb):
                bbs = jnp.full((L,), bb, jnp.int32)
                for kk in range(D // L):
                    v = emb[b][bb, pl.ds(L * kk, L)]
                    plsc.store_scatter(st[p], [dvecs[kk], bbs], v)

        def step(s, u, first, last):
            b, p = u % NBUF, u % NST
            if not last:
                fire_gather(s + 6, (u + 6) % NBUF)
            drain_gather(b)
            if not first:
                wait_out(p)
            transpose_scale(b, p)
            fire_out(s, p)

        for s in range(6):
            fire_gather(s, s)

        for u in range(NBUF):
            step(u, u, first=(u < NST), last=False)

        def rev(s8, carry):
            for u in range(NBUF):
                step(s8 * NBUF + u, u, first=False, last=False)
            return carry
        lax.fori_loop(1, S // NBUF - 1, rev, 0)

        base = (S // NBUF - 1) * NBUF
        for u in range(NBUF):
            step(base + u, u, first=False, last=(u >= 2))
        for p in range(NST):
            wait_out(p)

    return k


def _tc_prep_body(t_ref, o_ref):
    eye = jnp.eye(D, dtype=jnp.float32) * SCALE
    o_ref[:, 0:D] = lax.dot_general(
        t_ref[...], eye, (((0,), (0,)), ((), ())),
        preferred_element_type=jnp.float32)


def _tc_prep(tT):
    V = tT.shape[1]
    B2 = 512
    return pl.pallas_call(
        _tc_prep_body,
        out_shape=jax.ShapeDtypeStruct((V, DP), jnp.float32),
        grid=(pl.cdiv(V, B2),),
        in_specs=[pl.BlockSpec((D, B2), lambda g: (0, g))],
        out_specs=pl.BlockSpec((B2, DP), lambda g: (g, 0)),
    )(tT)


def kernel(x, table):
    B, S = x.shape
    V = table.shape[0]
    xt = (x.astype(jnp.int32) * 2).T
    tp = _tc_prep(table.T).reshape(2 * V, D)
    out5 = _build_sc_kernel(B, S, V)(xt, tp)
    return out5.transpose(2, 4, 0, 1, 3).reshape(B, S, D)


# speedup vs baseline: 2.0819x; 2.0819x over previous
import functools

import jax
import jax.numpy as jnp
from jax import lax
from jax.experimental import pallas as pl
from jax.experimental.pallas import tpu as pltpu
from jax.experimental.pallas import tpu_sc as plsc

D = 64
DP = 128
SP = 136
SCALE = 8.0
L = 16
NBUF = 8
NST = 4


def _build_sc_kernel(B: int, S: int, V: int):
    info = plsc.get_sparse_core_info()
    NW = info.num_cores * info.num_subcores
    CH = B // NW
    assert CH == 128 and S % NBUF == 0 and D % L == 0

    mesh = plsc.VectorSubcoreMesh(core_axis_name="c", subcore_axis_name="s")

    @functools.partial(
        pl.kernel,
        mesh=mesh,
        out_type=jax.ShapeDtypeStruct((S, D // 8, NW, 8, DP), jnp.float32),
        scratch_types=[
            pltpu.VMEM((S, CH), jnp.int32),
            *[pltpu.VMEM((CH, D), jnp.float32) for _ in range(NBUF)],
            *[pltpu.VMEM((D, SP), jnp.float32) for _ in range(NST)],
            *[pltpu.SemaphoreType.DMA for _ in range(NBUF + NST)],
        ],
        compiler_params=pltpu.CompilerParams(
            use_tc_tiling_on_sc=False, needs_layout_passes=False),
    )
    def k(xt_hbm, tp_hbm, out_hbm, idx_v, *refs):
        emb = refs[:NBUF]
        st = refs[NBUF:NBUF + NST]
        gsem = refs[NBUF + NST:2 * NBUF + NST]
        osem = refs[2 * NBUF + NST:]

        wid = lax.axis_index("s") * info.num_cores + lax.axis_index("c")

        pltpu.sync_copy(xt_hbm.at[pl.ds(0, S), pl.ds(wid * CH, CH)], idx_v)

        dvecs = [lax.iota(jnp.int32, L) + L * kk for kk in range(D // L)]

        def fire_gather(s, b):
            pltpu.async_copy(tp_hbm.at[idx_v.at[s]], emb[b], gsem[b])

        def drain_gather(b):
            pltpu.make_async_copy(tp_hbm.at[pl.ds(0, CH)], emb[b], gsem[b]).wait()

        def fire_out(s, p):
            for do in range(D // 8):
                pltpu.async_copy(
                    st[p].at[pl.ds(8 * do, 8), pl.ds(0, DP)],
                    out_hbm.at[s, do, wid], osem[p])

        def wait_out(p):
            for do in range(D // 8):
                pltpu.make_async_copy(
                    st[p].at[pl.ds(0, 8), pl.ds(0, DP)],
                    out_hbm.at[0, 0, 0], osem[p]).wait()

        def transpose_scale(b, p):
            @plsc.parallel_loop(0, CH, unroll=2)
            def body(bb):
                bbs = jnp.full((L,), bb, jnp.int32)
                for kk in range(D // L):
                    v = emb[b][bb, pl.ds(L * kk, L)] * SCALE
                    plsc.store_scatter(st[p], [dvecs[kk], bbs], v)

        def step(s, u, first, last):
            b, p = u % NBUF, u % NST
            if not last:
                fire_gather(s + 6, (u + 6) % NBUF)
            drain_gather(b)
            if not first:
                wait_out(p)
            transpose_scale(b, p)
            fire_out(s, p)

        for s in range(6):
            fire_gather(s, s)

        for u in range(NBUF):
            step(u, u, first=(u < NST), last=False)

        def rev(s8, carry):
            for u in range(NBUF):
                step(s8 * NBUF + u, u, first=False, last=False)
            return carry
        lax.fori_loop(1, S // NBUF - 1, rev, 0)

        base = (S // NBUF - 1) * NBUF
        for u in range(NBUF):
            step(base + u, u, first=False, last=(u >= 2))
        for p in range(NST):
            wait_out(p)

    return k


def _tc_prep_body(t_ref, o_ref):
    eye = jnp.eye(D, dtype=jnp.float32) * SCALE
    o_ref[:, 0:D] = lax.dot_general(
        t_ref[...], eye, (((0,), (0,)), ((), ())),
        preferred_element_type=jnp.float32)


def _tc_prep(tT):
    V = tT.shape[1]
    B2 = 512
    return pl.pallas_call(
        _tc_prep_body,
        out_shape=jax.ShapeDtypeStruct((V, DP), jnp.float32),
        grid=(pl.cdiv(V, B2),),
        in_specs=[pl.BlockSpec((D, B2), lambda g: (0, g))],
        out_specs=pl.BlockSpec((B2, DP), lambda g: (g, 0)),
    )(tT)


def kernel(x, table):
    B, S = x.shape
    V = table.shape[0]
    xt = (x.astype(jnp.int32) * 2).T
    tp = jnp.concatenate(
        [table, jnp.zeros((V, DP - D), jnp.float32)], axis=1).reshape(2 * V, D)
    out5 = _build_sc_kernel(B, S, V)(xt, tp)
    return out5.transpose(2, 4, 0, 1, 3).reshape(B, S, D)
